# baseline (speedup 1.0000x reference)
import jax
import jax.numpy as jnp
import numpy as np
from jax.experimental import pallas as pl
from jax.experimental.pallas import tpu as pltpu

C0, C1, C2 = 3, 32, 64
NFEAT = 16384
NHID = 128
NOUT = 11
NOUT_PAD = 128

K1B = 3 * C0 * 64 + 1
K2B = 32 * C1
N1 = 64 * C1
N2 = 32 * C2

_VMEM_LIMIT = 60 * 1024 * 1024


def _shift_dn(a, h):
    lo, hi = a[:, :, 0:h], a[:, :, h:2 * h]
    z = jnp.zeros_like(a[:, :, 0:1])
    return jnp.concatenate(
        [jnp.concatenate([z, hi[:, :, :h - 1]], axis=2), lo], axis=2)


def _shift_up(a, h):
    lo, hi = a[:, :, 0:h], a[:, :, h:2 * h]
    z = jnp.zeros_like(a[:, :, 0:1])
    return jnp.concatenate(
        [hi, jnp.concatenate([lo[:, :, 1:], z], axis=2)], axis=2)


def _perm_matrix():
    pm = np.zeros((64, 64), np.float32)
    for r in range(64):
        pp, p2, y4 = r // 32, (r // 16) % 2, r % 16
        pm[r, 4 * y4 + 2 * p2 + pp] = 1.0
    return pm


def _conv_tower_kernel(x_ref, pc_ref, w1_ref, w2_ref, b2_ref, o_ref):
    bt = x_ref.shape[0]
    xx = x_ref[...].astype(jnp.bfloat16)

    zrow = jnp.zeros((bt, 3, 1, 64), jnp.bfloat16)
    ym1 = jnp.concatenate([zrow, xx[:, :, :63]], axis=2)
    yp1 = jnp.concatenate([xx[:, :, 1:], zrow], axis=2)
    ones = jnp.ones((bt, 64, 1), jnp.bfloat16)
    lhs1n = jnp.concatenate(
        [ym1[:, 0], ym1[:, 1], ym1[:, 2],
         xx[:, 0], xx[:, 1], xx[:, 2],
         yp1[:, 0], yp1[:, 1], yp1[:, 2], ones], axis=-1)

    pc = pc_ref[...]
    lhs1 = jnp.stack(
        [jnp.dot(pc, lhs1n[b],
                 preferred_element_type=jnp.float32).astype(jnp.bfloat16)
         for b in range(bt)], axis=0)

    y1 = jnp.dot(lhs1.reshape(bt * 64, K1B), w1_ref[...],
                 preferred_element_type=jnp.float32)

    px = jnp.maximum(y1[:, :1024], y1[:, 1024:])
    px = px.reshape(bt, 2, 32, 1024)
    p1 = jnp.maximum(px[:, 0], px[:, 1])
    p1 = jnp.maximum(p1, 0.0).astype(jnp.bfloat16)

    p1 = p1.reshape(bt, 1, 32, K2B)
    lhs2 = jnp.concatenate(
        [_shift_dn(p1, 16)[:, 0], p1[:, 0], _shift_up(p1, 16)[:, 0]],
        axis=-1)
    y2 = jnp.dot(lhs2.reshape(bt * 32, 3 * K2B), w2_ref[...],
                 preferred_element_type=jnp.float32)

    qx = jnp.maximum(y2[:, :1024], y2[:, 1024:])
    qx = qx.reshape(bt, 2, 16, 1024)
    p2 = jnp.maximum(qx[:, 0], qx[:, 1])
    p2 = jnp.maximum(p2 + b2_ref[...], 0.0)

    o_ref[...] = p2.astype(jnp.bfloat16)


def _conv_tower(x, w1b, w2b, b2p, bt):
    B = x.shape[0]
    return pl.pallas_call(
        _conv_tower_kernel,
        out_shape=jax.ShapeDtypeStruct((B, 16, 1024), jnp.bfloat16),
        grid=(B // bt,),
        in_specs=[
            pl.BlockSpec((bt, 3, 64, 64), lambda i: (i, 0, 0, 0)),
            pl.BlockSpec((64, 64), lambda i: (0, 0)),
            pl.BlockSpec((K1B, N1), lambda i: (0, 0)),
            pl.BlockSpec((3 * K2B, N2), lambda i: (0, 0)),
            pl.BlockSpec((1, 1024), lambda i: (0, 0)),
        ],
        out_specs=pl.BlockSpec((bt, 16, 1024), lambda i: (i, 0, 0)),
        compiler_params=pltpu.CompilerParams(
            dimension_semantics=("parallel",),
            vmem_limit_bytes=_VMEM_LIMIT),
    )(x, jnp.asarray(_perm_matrix(), jnp.bfloat16), w1b, w2b, b2p)


def _mlp_kernel(x_ref, w1_ref, b1_ref, w2_ref, b2_ref, o_ref):
    h = jnp.dot(x_ref[...], w1_ref[...].astype(jnp.bfloat16),
                preferred_element_type=jnp.float32) + b1_ref[...]
    h = jnp.maximum(h, 0.0).astype(jnp.bfloat16)
    o_ref[...] = (jnp.dot(h, w2_ref[...].astype(jnp.bfloat16),
                          preferred_element_type=jnp.float32) + b2_ref[...])


def _mlp(x, w1, b1, w2p, b2p, bm):
    B, K = x.shape
    return pl.pallas_call(
        _mlp_kernel,
        out_shape=jax.ShapeDtypeStruct((B, NOUT_PAD), jnp.float32),
        grid=(B // bm,),
        in_specs=[
            pl.BlockSpec((bm, K), lambda i: (i, 0)),
            pl.BlockSpec((K, NHID), lambda i: (0, 0)),
            pl.BlockSpec((1, NHID), lambda i: (0, 0)),
            pl.BlockSpec((NHID, NOUT_PAD), lambda i: (0, 0)),
            pl.BlockSpec((1, NOUT_PAD), lambda i: (0, 0)),
        ],
        out_specs=pl.BlockSpec((bm, NOUT_PAD), lambda i: (i, 0)),
        compiler_params=pltpu.CompilerParams(
            dimension_semantics=("parallel",),
            vmem_limit_bytes=_VMEM_LIMIT),
    )(x, w1, b1, w2p, b2p)


def _band(nx, dtype=np.float32):
    b = np.zeros((3, nx, nx), dtype)
    for dx in range(3):
        for xo in range(nx):
            xi = xo + dx - 1
            if 0 <= xi < nx:
                b[dx, xi, xo] = 1.0
    return b


def _parity(nx):
    return np.concatenate([np.arange(0, nx, 2), np.arange(1, nx, 2)])


def _build_w1b(w1c, b1):
    w1r = w1c.reshape(3, 3, C0, C1).astype(jnp.bfloat16)
    t = jnp.einsum('YDIC,DXO->YIXOC', w1r,
                   jnp.asarray(_band(64, np.float32)).astype(jnp.bfloat16),
                   preferred_element_type=jnp.bfloat16)
    t = t[:, :, :, _parity(64), :]
    main = t.reshape(9 * 64, N1)
    return jnp.concatenate(
        [main, jnp.tile(b1, (1, 64)).astype(jnp.bfloat16)], axis=0)


def _build_w2b(w2d):
    w2r = w2d.reshape(3, 3, C1, C2).astype(jnp.bfloat16)
    t = jnp.einsum('YDIC,DXO->YXIOC', w2r,
                   jnp.asarray(_band(32, np.float32)).astype(jnp.bfloat16),
                   preferred_element_type=jnp.bfloat16)
    t = t[:, :, :, _parity(32), :]
    return t.reshape(3 * K2B, N2)


@jax.jit
def _forward(x, w1c, b1, w2d, b2, wf1, bf1, wf2p, bf2p):
    B = x.shape[0]

    w1b = _build_w1b(w1c, b1)
    w2b = _build_w2b(w2d)
    b2p = jnp.tile(b2, (1, 16))

    feats = _conv_tower(x, w1b, w2b, b2p, bt=min(8, B))
    feats = feats.reshape(B, NFEAT)
    logits = _mlp(feats, wf1, bf1, wf2p, bf2p, bm=min(64, B))
    return logits[:, :NOUT]


def kernel(x, w1c, b1, w2d, b2, wf1, bf1, wf2p, bf2p):
    return _forward(x, w1c, b1, w2d, b2, wf1, bf1, wf2p, bf2p)


# speedup vs baseline: 1.0380x; 1.0380x over previous
import jax
import jax.numpy as jnp
import numpy as np
from jax.experimental import pallas as pl
from jax.experimental.pallas import tpu as pltpu

C0, C1, C2 = 3, 32, 64
NFEAT = 16384
NHID = 128
NOUT = 11
NOUT_PAD = 128

K1B = 3 * C0 * 64 + 1
K2B = 32 * C1
N1 = 64 * C1
N2 = 32 * C2

_VMEM_LIMIT = 60 * 1024 * 1024


def _shift_dn(a, h):
    lo, hi = a[:, :, 0:h], a[:, :, h:2 * h]
    z = jnp.zeros_like(a[:, :, 0:1])
    return jnp.concatenate(
        [jnp.concatenate([z, hi[:, :, :h - 1]], axis=2), lo], axis=2)


def _shift_up(a, h):
    lo, hi = a[:, :, 0:h], a[:, :, h:2 * h]
    z = jnp.zeros_like(a[:, :, 0:1])
    return jnp.concatenate(
        [hi, jnp.concatenate([lo[:, :, 1:], z], axis=2)], axis=2)


def _perm_matrix():
    pm = np.zeros((64, 64), np.float32)
    for r in range(64):
        pp, p2, y4 = r // 32, (r // 16) % 2, r % 16
        pm[r, 4 * y4 + 2 * p2 + pp] = 1.0
    return pm


def _conv_tower_kernel(x_ref, w1_ref, w2_ref, b2_ref, o_ref):
    bt = x_ref.shape[0]
    xx = x_ref[...]

    xh0, xh1 = xx[:, :, 0:32], xx[:, :, 32:64]
    ym1 = jnp.concatenate([_shift_dn(xh1, 16), xh0], axis=2)
    yp1 = jnp.concatenate([xh1, _shift_up(xh0, 16)], axis=2)
    ones = jnp.ones((bt, 64, 1), jnp.bfloat16)
    lhs1 = jnp.concatenate(
        [ym1[:, 0], ym1[:, 1], ym1[:, 2],
         xx[:, 0], xx[:, 1], xx[:, 2],
         yp1[:, 0], yp1[:, 1], yp1[:, 2], ones], axis=-1)

    y1 = jnp.dot(lhs1.reshape(bt * 64, K1B), w1_ref[...],
                 preferred_element_type=jnp.float32)

    px = jnp.maximum(y1[:, :1024], y1[:, 1024:])
    px = px.reshape(bt, 2, 32, 1024)
    p1 = jnp.maximum(px[:, 0], px[:, 1])
    p1 = jnp.maximum(p1, 0.0).astype(jnp.bfloat16)

    p1 = p1.reshape(bt, 1, 32, K2B)
    lhs2 = jnp.concatenate(
        [_shift_dn(p1, 16)[:, 0], p1[:, 0], _shift_up(p1, 16)[:, 0]],
        axis=-1)
    y2 = jnp.dot(lhs2.reshape(bt * 32, 3 * K2B), w2_ref[...],
                 preferred_element_type=jnp.float32)

    qx = jnp.maximum(y2[:, :1024], y2[:, 1024:])
    qx = qx.reshape(bt, 2, 16, 1024)
    p2 = jnp.maximum(qx[:, 0], qx[:, 1])
    p2 = jnp.maximum(p2 + b2_ref[...], 0.0)

    o_ref[...] = p2.astype(jnp.bfloat16)


def _conv_tower(x, w1b, w2b, b2p, bt):
    B = x.shape[0]
    return pl.pallas_call(
        _conv_tower_kernel,
        out_shape=jax.ShapeDtypeStruct((B, 16, 1024), jnp.bfloat16),
        grid=(B // bt,),
        in_specs=[
            pl.BlockSpec((bt, 3, 64, 64), lambda i: (i, 0, 0, 0)),
            pl.BlockSpec((K1B, N1), lambda i: (0, 0)),
            pl.BlockSpec((3 * K2B, N2), lambda i: (0, 0)),
            pl.BlockSpec((1, 1024), lambda i: (0, 0)),
        ],
        out_specs=pl.BlockSpec((bt, 16, 1024), lambda i: (i, 0, 0)),
        compiler_params=pltpu.CompilerParams(
            dimension_semantics=("parallel",),
            vmem_limit_bytes=_VMEM_LIMIT),
    )(x, w1b, w2b, b2p)


def _mlp_kernel(x_ref, w1_ref, b1_ref, w2_ref, b2_ref, o_ref):
    h = jnp.dot(x_ref[...], w1_ref[...],
                preferred_element_type=jnp.float32) + b1_ref[...]
    h = jnp.maximum(h, 0.0).astype(jnp.bfloat16)
    o_ref[...] = (jnp.dot(h, w2_ref[...],
                          preferred_element_type=jnp.float32) + b2_ref[...])


def _mlp(x, w1, b1, w2p, b2p, bm):
    B, K = x.shape
    return pl.pallas_call(
        _mlp_kernel,
        out_shape=jax.ShapeDtypeStruct((B, NOUT_PAD), jnp.float32),
        grid=(B // bm,),
        in_specs=[
            pl.BlockSpec((bm, K), lambda i: (i, 0)),
            pl.BlockSpec((K, NHID), lambda i: (0, 0)),
            pl.BlockSpec((1, NHID), lambda i: (0, 0)),
            pl.BlockSpec((NHID, NOUT_PAD), lambda i: (0, 0)),
            pl.BlockSpec((1, NOUT_PAD), lambda i: (0, 0)),
        ],
        out_specs=pl.BlockSpec((bm, NOUT_PAD), lambda i: (i, 0)),
        compiler_params=pltpu.CompilerParams(
            dimension_semantics=("parallel",),
            vmem_limit_bytes=_VMEM_LIMIT),
    )(x, w1, b1, w2p, b2p)


def _band(nx, dtype=np.float32):
    b = np.zeros((3, nx, nx), dtype)
    for dx in range(3):
        for xo in range(nx):
            xi = xo + dx - 1
            if 0 <= xi < nx:
                b[dx, xi, xo] = 1.0
    return b


def _parity(nx):
    return np.concatenate([np.arange(0, nx, 2), np.arange(1, nx, 2)])


def _build_w1b(w1c, b1):
    w1r = w1c.reshape(3, 3, C0, C1).astype(jnp.bfloat16)
    t = jnp.einsum('YDIC,DXO->YIXOC', w1r,
                   jnp.asarray(_band(64, np.float32)).astype(jnp.bfloat16),
                   preferred_element_type=jnp.bfloat16)
    t = t[:, :, :, _parity(64), :]
    main = t.reshape(9 * 64, N1)
    return jnp.concatenate(
        [main, jnp.tile(b1, (1, 64)).astype(jnp.bfloat16)], axis=0)


def _build_w2b(w2d):
    w2r = w2d.reshape(3, 3, C1, C2).astype(jnp.bfloat16)
    t = jnp.einsum('YDIC,DXO->YXIOC', w2r,
                   jnp.asarray(_band(32, np.float32)).astype(jnp.bfloat16),
                   preferred_element_type=jnp.bfloat16)
    t = t[:, :, :, _parity(32), :]
    return t.reshape(3 * K2B, N2)


@jax.jit
def _forward(x, w1c, b1, w2d, b2, wf1, bf1, wf2p, bf2p):
    B = x.shape[0]
    perm = np.array([4 * y4 + 2 * p2 + p
                     for p in (0, 1) for p2 in (0, 1) for y4 in range(16)])
    xpp = x.astype(jnp.bfloat16)[:, :, perm]

    w1b = _build_w1b(w1c, b1)
    w2b = _build_w2b(w2d)
    b2p = jnp.tile(b2, (1, 16))

    feats = _conv_tower(xpp, w1b, w2b, b2p, bt=min(8, B))
    feats = feats.reshape(B, NFEAT)
    logits = _mlp(feats, wf1.astype(jnp.bfloat16), bf1,
                  wf2p.astype(jnp.bfloat16), bf2p, bm=min(64, B))
    return logits[:, :NOUT]


def kernel(x, w1c, b1, w2d, b2, wf1, bf1, wf2p, bf2p):
    return _forward(x, w1c, b1, w2d, b2, wf1, bf1, wf2p, bf2p)
